# baseline (speedup 1.0000x reference)
import functools

import jax
import jax.numpy as jnp
from jax import lax
from jax.experimental import pallas as pl
from jax.experimental.pallas import tpu as pltpu
from jax.experimental.pallas import tpu_sc as plsc

N = 100000
E = 6400000
D = 128
N_PAD = 100352
SUB = 112
LPC = 19
CHUNK = 2048
NCHUNK = 100
EPT = CHUNK * NCHUNK
E_PAD = 32 * EPT
TSLICE = N_PAD // 16
NCB = 49

_mesh = lambda: plsc.VectorSubcoreMesh(core_axis_name="c", subcore_axis_name="s")


def _wid():
    return lax.axis_index("c") * 16 + lax.axis_index("s")


def _embed(x, x3d, w_exp, scal):
    @functools.partial(
        pl.kernel,
        mesh=_mesh(),
        compiler_params=pltpu.CompilerParams(needs_layout_passes=False),
        out_type=jax.ShapeDtypeStruct((N_PAD,), jnp.float32),
        scratch_types=[
            pltpu.VMEM((128, 128), jnp.float32),
            pltpu.VMEM((3456,), jnp.float32),
            pltpu.VMEM((16 * D,), jnp.float32),
            pltpu.VMEM((16,), jnp.float32),
        ],
    )
    def body(x_hbm, x3_hbm, w_hbm, scal_hbm, out_hbm, xbuf, hbuf, w_v, scal_v):
        wid = _wid()
        pltpu.sync_copy(w_hbm, w_v)
        pltpu.sync_copy(scal_hbm, scal_v)
        sv = scal_v[pl.ds(0, 16)]
        b_e = sv[0]
        w_c = sv[1]
        wbase = jnp.where(wid < 13, wid * 25, 325 + (wid - 13) * 24)
        n_grp = jnp.where(wid < 13, 200, 192)
        lane16 = lax.iota(jnp.int32, 16)

        def dot16(g):
            grows = g * 16 + lane16
            acc0 = jnp.zeros((16,), jnp.float32)
            acc1 = jnp.zeros((16,), jnp.float32)
            acc2 = jnp.zeros((16,), jnp.float32)
            acc3 = jnp.zeros((16,), jnp.float32)
            for k in range(0, D, 4):
                c0 = jnp.full((16,), k, jnp.int32)
                acc0 = acc0 + plsc.load_gather(xbuf, [grows, c0]) * w_v[pl.ds(k * 16, 16)]
                acc1 = acc1 + plsc.load_gather(xbuf, [grows, c0 + 1]) * w_v[pl.ds((k + 1) * 16, 16)]
                acc2 = acc2 + plsc.load_gather(xbuf, [grows, c0 + 2]) * w_v[pl.ds((k + 2) * 16, 16)]
                acc3 = acc3 + plsc.load_gather(xbuf, [grows, c0 + 3]) * w_v[pl.ds((k + 3) * 16, 16)]
            h16 = ((acc0 + acc1) + (acc2 + acc3)) + b_e
            return jnp.maximum(h16, 0.0) * w_c

        def grp(i, carry):
            w = i // 8
            g = i - w * 8

            @pl.when(g == 0)
            def _():
                pltpu.sync_copy(x3_hbm.at[wbase + w], xbuf)

            hbuf[pl.ds(w * 128 + g * 16, 16)] = dot16(g)
            return carry

        lax.fori_loop(0, n_grp, grp, 0)

        @pl.when(wid == 31)
        def _():
            pltpu.sync_copy(x_hbm.at[pl.ds(N - 128, 128), :], xbuf)
            hbuf[pl.ds(24 * 128 + 0 * 16, 16)] = dot16(6)
            hbuf[pl.ds(24 * 128 + 1 * 16, 16)] = dot16(7)

        @pl.when(wid < 13)
        def _():
            pltpu.sync_copy(hbuf.at[pl.ds(0, 3200)],
                            out_hbm.at[pl.ds(wid * 3200, 3200)])

        @pl.when(jnp.logical_and(wid >= 13, wid < 31))
        def _():
            pltpu.sync_copy(hbuf.at[pl.ds(0, 3072)],
                            out_hbm.at[pl.ds(41600 + (wid - 13) * 3072, 3072)])

        @pl.when(wid == 31)
        def _():
            pltpu.sync_copy(hbuf.at[pl.ds(0, 3456)],
                            out_hbm.at[pl.ds(96896, 3456)])

    return body(x, x3d, w_exp, scal)


def _round(src3, dst3, h_or_agg, scal, first):
    @functools.partial(
        pl.kernel,
        mesh=_mesh(),
        compiler_params=pltpu.CompilerParams(needs_layout_passes=False),
        out_type=jax.ShapeDtypeStruct((2 * N_PAD,), jnp.float32),
        scratch_types=[
            pltpu.VMEM((N_PAD,), jnp.float32),
            pltpu.VMEM((16, 128), jnp.int32),
            pltpu.VMEM((16, 128), jnp.int32),
            pltpu.VMEM((16, 128), jnp.int32),
            pltpu.VMEM((16, 128), jnp.int32),
            pltpu.VMEM((2 * LPC, SUB), jnp.int32),
            pltpu.VMEM((2 * LPC, SUB), jnp.float32),
            pltpu.VMEM((8, 128), jnp.float32),
            pltpu.VMEM((8, 128), jnp.float32),
            pltpu.VMEM((8, 128), jnp.float32),
            pltpu.VMEM((8, 128), jnp.float32),
            pltpu.VMEM((16,), jnp.float32),
            pltpu.VMEM((1024,), jnp.float32),
            pltpu.VMEM_SHARED((N_PAD,), jnp.float32),
            pltpu.SemaphoreType.DMA,
            pltpu.SemaphoreType.DMA,
            pltpu.SemaphoreType.DMA,
            pltpu.SemaphoreType.DMA,
        ],
    )
    def body(src_hbm, dst_hbm, hin_hbm, scal_hbm, out_hbm,
             h_v, srcbA, srcbB, dstgA, dstgB, dstb, valsb,
             a0A, a1A, a0B, a1B, scal_v, zbuf,
             agg_sp, semA, semB, scA, scB):
        cid = lax.axis_index("c")
        sid = lax.axis_index("s")
        wid = cid * 16 + sid
        pltpu.sync_copy(scal_hbm, scal_v)
        sv = scal_v[pl.ds(0, 16)]
        b_c = sv[2]
        w_c = sv[1]

        if first:
            pltpu.sync_copy(hin_hbm, h_v)
        else:
            def c_start(c, b0, b1, sem):
                pltpu.async_copy(hin_hbm.at[c], b0, sem)
                pltpu.async_copy(hin_hbm.at[98 + c], b1, sem)

            def c_wait(b0, b1, sem):
                pltpu.make_async_copy(hin_hbm.at[0], b0, sem).wait()
                pltpu.make_async_copy(hin_hbm.at[0], b1, sem).wait()

            def c_compute(c, b0, b1):
                for j in range(64):
                    v = (b0[j // 8, pl.ds((j % 8) * 16, 16)]
                         + b1[j // 8, pl.ds((j % 8) * 16, 16)] + b_c)
                    h_v[pl.ds(c * 1024 + j * 16, 16)] = jnp.maximum(v, 0.0) * w_c

            c_start(0, a0A, a1A, semA)

            def comb(q, carry):
                ca = 2 * q
                cb = 2 * q + 1
                c_start(cb, a0B, a1B, semB)
                c_wait(a0A, a1A, semA)
                c_compute(ca, a0A, a1A)

                @pl.when(ca + 2 < 98)
                def _():
                    c_start(ca + 2, a0A, a1A, semA)
                c_wait(a0B, a1B, semB)
                c_compute(cb, a0B, a1B)
                return carry

            lax.fori_loop(0, 49, comb, 0)

        z = jnp.zeros((16,), jnp.float32)
        for j in range(64):
            zbuf[pl.ds(j * 16, 16)] = z
        for t in range(6):
            pltpu.sync_copy(zbuf, agg_sp.at[pl.ds((sid * 6 + t) * 1024, 1024)])

        @pl.when(sid == 15)
        def _():
            pltpu.sync_copy(zbuf, agg_sp.at[pl.ds(96 * 1024, 1024)])
            pltpu.sync_copy(zbuf, agg_sp.at[pl.ds(97 * 1024, 1024)])
        for t in range(5):
            dstb[18, pl.ds(32 + t * 16, 16)] = jnp.zeros((16,), jnp.int32)
            dstb[LPC + 18, pl.ds(32 + t * 16, 16)] = jnp.zeros((16,), jnp.int32)
            valsb[18, pl.ds(32 + t * 16, 16)] = z
            valsb[LPC + 18, pl.ds(32 + t * 16, 16)] = z
        plsc.subcore_barrier()

        cbase = wid * NCHUNK

        def e_start(c, sb, dg, sem):
            pltpu.async_copy(src_hbm.at[cbase + c], sb, sem)
            pltpu.async_copy(dst_hbm.at[cbase + c], dg, sem)

        def e_wait(sb, dg, sem):
            pltpu.make_async_copy(src_hbm.at[0], sb, sem).wait()
            pltpu.make_async_copy(dst_hbm.at[0], dg, sem).wait()

        def gather(sb, dg, half):
            for j in range(128):
                v = plsc.load_gather(h_v, [sb[j // 8, pl.ds((j % 8) * 16, 16)]])
                p = j * 16
                r, col = p // SUB, p % SUB
                valsb[half * LPC + r, pl.ds(col, 16)] = v
                dstb[half * LPC + r, pl.ds(col, 16)] = dg[j // 8, pl.ds((j % 8) * 16, 16)]

        def sc_issue(half, sem):
            for u in range(LPC):
                pltpu.async_copy(valsb.at[half * LPC + u],
                                 agg_sp.at[dstb.at[half * LPC + u]], sem, add=True)

        def sc_drain(half, sem):
            for u in range(LPC):
                pltpu.make_async_copy(valsb.at[half * LPC + u],
                                      agg_sp.at[dstb.at[half * LPC + u]], sem).wait()

        e_start(0, srcbA, dstgA, semA)

        def chunk(q, carry):
            a = 2 * q

            @pl.when(q > 0)
            def _():
                sc_drain(1, scB)
            e_start(a + 1, srcbB, dstgB, semB)
            e_wait(srcbA, dstgA, semA)
            gather(srcbA, dstgA, 0)
            sc_issue(0, scA)
            e_wait(srcbB, dstgB, semB)
            gather(srcbB, dstgB, 1)
            sc_drain(0, scA)

            @pl.when(a + 2 < NCHUNK)
            def _():
                e_start(a + 2, srcbA, dstgA, semA)
            sc_issue(1, scB)
            return carry

        lax.fori_loop(0, NCHUNK // 2, chunk, 0)
        sc_drain(1, scB)
        plsc.subcore_barrier()
        pltpu.sync_copy(agg_sp.at[pl.ds(sid * TSLICE, TSLICE)],
                        out_hbm.at[pl.ds(cid * N_PAD + sid * TSLICE, TSLICE)])

    return body(src3, dst3, h_or_agg, scal)


def _finalize(agg, scal):
    @functools.partial(
        pl.kernel,
        mesh=_mesh(),
        compiler_params=pltpu.CompilerParams(needs_layout_passes=False),
        out_type=jax.ShapeDtypeStruct((N_PAD,), jnp.float32),
        scratch_types=[
            pltpu.VMEM((3136,), jnp.float32),
            pltpu.VMEM((3136,), jnp.float32),
            pltpu.VMEM((16,), jnp.float32),
        ],
    )
    def body(agg_hbm, scal_hbm, out_hbm, a0b, a1b, scal_v):
        wid = _wid()
        pltpu.sync_copy(scal_hbm, scal_v)
        sv = scal_v[pl.ds(0, 16)]
        b_c = sv[2]
        base = wid * 3136
        pltpu.sync_copy(agg_hbm.at[pl.ds(base, 3136)], a0b)
        pltpu.sync_copy(agg_hbm.at[pl.ds(N_PAD + base, 3136)], a1b)
        for j in range(196):
            v = a0b[pl.ds(j * 16, 16)] + a1b[pl.ds(j * 16, 16)] + b_c
            a0b[pl.ds(j * 16, 16)] = jnp.maximum(v, 0.0)
        pltpu.sync_copy(a0b, out_hbm.at[pl.ds(base, 3136)])

    return body(agg, scal)


def kernel(x, edge_index, W_embed, b_embed, W_conv, b_conv):
    src = jnp.concatenate([edge_index[0], jnp.zeros((E_PAD - E,), jnp.int32)])
    dst = jnp.concatenate([edge_index[1],
                           jnp.full((E_PAD - E,), N, jnp.int32)])
    src3 = src.reshape(E_PAD // CHUNK, 16, 128)
    dst3 = dst.reshape(E_PAD // CHUNK, 16, 128)
    x3d = x[:99968].reshape(781, 128, 128)
    w_exp = jnp.repeat(W_embed.reshape(-1).astype(jnp.float32), 16)
    scal = jnp.concatenate([
        b_embed.reshape(-1), W_conv.reshape(-1), b_conv.reshape(-1),
        jnp.zeros((13,), jnp.float32),
    ]).astype(jnp.float32)
    h0 = _embed(x, x3d, w_exp, scal)
    agg = _round(src3, dst3, h0, scal, first=True)
    for _ in range(3):
        agg = _round(src3, dst3, agg.reshape(196, 8, 128), scal,
                     first=False)
    hf = _finalize(agg, scal)
    return hf[:N].reshape(N, 1)


# speedup vs baseline: 1.4790x; 1.4790x over previous
import functools

import jax
import jax.numpy as jnp
from jax import lax
from jax.experimental import pallas as pl
from jax.experimental.pallas import tpu as pltpu
from jax.experimental.pallas import tpu_sc as plsc

N = 100000
E = 6400000
D = 128
N_PAD = 100352
ROWS_LO = 3136
ROWS_LAST = N - 31 * ROWS_LO
SUB = 112
SPC = 16
CHUNK = SUB * SPC
NCHUNK = 112
EPT = CHUNK * NCHUNK
E_PAD = 32 * EPT
TSLICE = N_PAD // 16
CB = 2000

_mesh = lambda: plsc.VectorSubcoreMesh(core_axis_name="c", subcore_axis_name="s")


def _wid():
    return lax.axis_index("c") * 16 + lax.axis_index("s")


def _embed(x_flat, w_exp, scal):
    @functools.partial(
        pl.kernel,
        mesh=_mesh(),
        compiler_params=pltpu.CompilerParams(needs_layout_passes=False),
        out_type=jax.ShapeDtypeStruct((N,), jnp.float32),
        scratch_types=[
            pltpu.VMEM((128, 128), jnp.float32),
            pltpu.VMEM((ROWS_LO,), jnp.float32),
            pltpu.VMEM((16 * D,), jnp.float32),
            pltpu.VMEM((16,), jnp.float32),
        ],
    )
    def body(x_hbm, w_hbm, scal_hbm, out_hbm, xbuf, hbuf, w_v, scal_v):
        wid = _wid()
        pltpu.sync_copy(w_hbm, w_v)
        pltpu.sync_copy(scal_hbm, scal_v)
        sv = scal_v[pl.ds(0, 16)]
        b_e = sv[0]
        w_c = sv[1]
        rows = jnp.where(wid < 31, ROWS_LO, ROWS_LAST)
        base = wid * ROWS_LO
        n_grp = jnp.where(wid < 31, 200, 176)
        lane16 = lax.iota(jnp.int32, 16)

        def grp(i, carry):
            w = i // 8
            g = i - w * 8
            sw = jnp.minimum(w * 128, rows - 128)

            @pl.when(g == 0)
            def _():
                pltpu.sync_copy(x_hbm.at[pl.ds(base + sw, 128), :], xbuf)

            grows = g * 16 + lane16
            acc0 = jnp.zeros((16,), jnp.float32)
            acc1 = jnp.zeros((16,), jnp.float32)
            acc2 = jnp.zeros((16,), jnp.float32)
            acc3 = jnp.zeros((16,), jnp.float32)
            for k in range(0, D, 4):
                c0 = jnp.full((16,), k, jnp.int32)
                acc0 = acc0 + plsc.load_gather(xbuf, [grows, c0]) * w_v[pl.ds(k * 16, 16)]
                acc1 = acc1 + plsc.load_gather(xbuf, [grows, c0 + 1]) * w_v[pl.ds((k + 1) * 16, 16)]
                acc2 = acc2 + plsc.load_gather(xbuf, [grows, c0 + 2]) * w_v[pl.ds((k + 2) * 16, 16)]
                acc3 = acc3 + plsc.load_gather(xbuf, [grows, c0 + 3]) * w_v[pl.ds((k + 3) * 16, 16)]
            h16 = ((acc0 + acc1) + (acc2 + acc3)) + b_e
            h16 = jnp.maximum(h16, 0.0) * w_c
            hbuf[pl.ds(sw + g * 16, 16)] = h16
            return carry

        lax.fori_loop(0, n_grp, grp, 0)

        @pl.when(wid < 31)
        def _():
            pltpu.sync_copy(hbuf, out_hbm.at[pl.ds(base, ROWS_LO)])

        @pl.when(wid == 31)
        def _():
            pltpu.sync_copy(hbuf.at[pl.ds(0, ROWS_LAST)],
                            out_hbm.at[pl.ds(base, ROWS_LAST)])

    return body(x_flat, w_exp, scal)


def _round(src, dst3d, h_or_agg, scal, first):
    @functools.partial(
        pl.kernel,
        mesh=_mesh(),
        compiler_params=pltpu.CompilerParams(needs_layout_passes=False),
        out_type=jax.ShapeDtypeStruct((2 * N_PAD,), jnp.float32),
        scratch_types=[
            pltpu.VMEM((N,), jnp.float32),
            pltpu.VMEM((2 * CHUNK,), jnp.int32),
            pltpu.VMEM((2 * SPC, SUB), jnp.int32),
            pltpu.VMEM((2 * SPC, SUB), jnp.float32),
            pltpu.VMEM((2 * CB,), jnp.float32),
            pltpu.VMEM((2 * CB,), jnp.float32),
            pltpu.VMEM((16,), jnp.float32),
            pltpu.VMEM_SHARED((N_PAD,), jnp.float32),
            pltpu.SemaphoreType.DMA,
            pltpu.SemaphoreType.DMA,
            pltpu.SemaphoreType.DMA,
            pltpu.SemaphoreType.DMA,
        ],
    )
    def body(src_hbm, dst_hbm, hin_hbm, scal_hbm, out_hbm,
             h_v, srcb, dstb, valsb, a0b, a1b, scal_v,
             agg_sp, semA, semB, scA, scB):
        cid = lax.axis_index("c")
        sid = lax.axis_index("s")
        wid = cid * 16 + sid
        pltpu.sync_copy(scal_hbm, scal_v)
        sv = scal_v[pl.ds(0, 16)]
        b_c = sv[2]
        w_c = sv[1]

        if first:
            pltpu.sync_copy(hin_hbm, h_v)
        else:
            def c_start(c, half, sem):
                pltpu.async_copy(hin_hbm.at[pl.ds(c * CB, CB)],
                                 a0b.at[pl.ds(half * CB, CB)], sem)
                pltpu.async_copy(hin_hbm.at[pl.ds(N_PAD + c * CB, CB)],
                                 a1b.at[pl.ds(half * CB, CB)], sem)

            def c_wait(half, sem):
                pltpu.make_async_copy(hin_hbm.at[pl.ds(0, CB)],
                                      a0b.at[pl.ds(half * CB, CB)], sem).wait()
                pltpu.make_async_copy(hin_hbm.at[pl.ds(0, CB)],
                                      a1b.at[pl.ds(half * CB, CB)], sem).wait()

            def c_compute(c, half):
                for j in range(CB // 16):
                    v = (a0b[pl.ds(half * CB + j * 16, 16)]
                         + a1b[pl.ds(half * CB + j * 16, 16)] + b_c)
                    h_v[pl.ds(c * CB + j * 16, 16)] = jnp.maximum(v, 0.0) * w_c

            c_start(0, 0, semA)

            def comb(q, carry):
                ca = 2 * q
                cb = 2 * q + 1
                c_start(cb, 1, semB)
                c_wait(0, semA)
                c_compute(ca, 0)

                @pl.when(ca + 2 < 50)
                def _():
                    c_start(ca + 2, 0, semA)
                c_wait(1, semB)
                c_compute(cb, 1)
                return carry

            lax.fori_loop(0, 25, comb, 0)

        z = jnp.zeros((16,), jnp.float32)
        for j in range(2 * CB // 16):
            a0b[pl.ds(j * 16, 16)] = z
        pltpu.sync_copy(a0b, agg_sp.at[pl.ds(sid * TSLICE, 2 * CB)])
        pltpu.sync_copy(a0b.at[pl.ds(0, TSLICE - 2 * CB)],
                        agg_sp.at[pl.ds(sid * TSLICE + 2 * CB, TSLICE - 2 * CB)])
        plsc.subcore_barrier()

        ebase = wid * EPT
        rbase = wid * (EPT // SUB)

        def e_start(c, half, sem):
            pltpu.async_copy(src_hbm.at[pl.ds(ebase + c * CHUNK, CHUNK)],
                             srcb.at[pl.ds(half * CHUNK, CHUNK)], sem)
            pltpu.async_copy(dst_hbm.at[pl.ds(rbase + c * SPC, SPC), :],
                             dstb.at[pl.ds(half * SPC, SPC), :], sem)

        def e_wait(half, sem):
            pltpu.make_async_copy(src_hbm.at[pl.ds(0, CHUNK)],
                                  srcb.at[pl.ds(half * CHUNK, CHUNK)], sem).wait()
            pltpu.make_async_copy(dst_hbm.at[pl.ds(0, SPC), :],
                                  dstb.at[pl.ds(half * SPC, SPC), :], sem).wait()

        def gather(half):
            for j in range(CHUNK // 16):
                v = plsc.load_gather(h_v, [srcb[pl.ds(half * CHUNK + j * 16, 16)]])
                valsb[half * SPC + j // 7, pl.ds((j % 7) * 16, 16)] = v

        def sc_issue(half, sem):
            for u in range(SPC):
                pltpu.async_copy(valsb.at[half * SPC + u],
                                 agg_sp.at[dstb.at[half * SPC + u]], sem, add=True)

        def sc_drain(half, sem):
            for u in range(SPC):
                pltpu.make_async_copy(valsb.at[half * SPC + u],
                                      agg_sp.at[dstb.at[half * SPC + u]], sem).wait()

        e_start(0, 0, semA)

        def chunk(q, carry):
            a = 2 * q

            @pl.when(q > 0)
            def _():
                sc_drain(1, scB)
            e_start(a + 1, 1, semB)
            e_wait(0, semA)
            gather(0)
            sc_issue(0, scA)
            e_wait(1, semB)
            gather(1)
            sc_drain(0, scA)

            @pl.when(a + 2 < NCHUNK)
            def _():
                e_start(a + 2, 0, semA)
            sc_issue(1, scB)
            return carry

        lax.fori_loop(0, NCHUNK // 2, chunk, 0)
        sc_drain(1, scB)
        plsc.subcore_barrier()
        pltpu.sync_copy(agg_sp.at[pl.ds(sid * TSLICE, TSLICE)],
                        out_hbm.at[pl.ds(cid * N_PAD + sid * TSLICE, TSLICE)])

    return body(src, dst3d, h_or_agg, scal)


def _finalize(agg, scal):
    @functools.partial(
        pl.kernel,
        mesh=_mesh(),
        compiler_params=pltpu.CompilerParams(needs_layout_passes=False),
        out_type=jax.ShapeDtypeStruct((N_PAD,), jnp.float32),
        scratch_types=[
            pltpu.VMEM((ROWS_LO,), jnp.float32),
            pltpu.VMEM((ROWS_LO,), jnp.float32),
            pltpu.VMEM((16,), jnp.float32),
        ],
    )
    def body(agg_hbm, scal_hbm, out_hbm, a0b, a1b, scal_v):
        wid = _wid()
        pltpu.sync_copy(scal_hbm, scal_v)
        sv = scal_v[pl.ds(0, 16)]
        b_c = sv[2]
        base = wid * ROWS_LO
        pltpu.sync_copy(agg_hbm.at[pl.ds(base, ROWS_LO)], a0b)
        pltpu.sync_copy(agg_hbm.at[pl.ds(N_PAD + base, ROWS_LO)], a1b)
        for j in range(ROWS_LO // 16):
            v = a0b[pl.ds(j * 16, 16)] + a1b[pl.ds(j * 16, 16)] + b_c
            a0b[pl.ds(j * 16, 16)] = jnp.maximum(v, 0.0)
        pltpu.sync_copy(a0b, out_hbm.at[pl.ds(base, ROWS_LO)])

    return body(agg, scal)


def kernel(x, edge_index, W_embed, b_embed, W_conv, b_conv):
    src = jnp.concatenate([edge_index[0], jnp.zeros((E_PAD - E,), jnp.int32)])
    dst = jnp.concatenate([edge_index[1],
                           jnp.full((E_PAD - E,), N, jnp.int32)])
    dst3d = dst.reshape(E_PAD // SUB, SUB)
    w_exp = jnp.repeat(W_embed.reshape(-1).astype(jnp.float32), 16)
    scal = jnp.concatenate([
        b_embed.reshape(-1), W_conv.reshape(-1), b_conv.reshape(-1),
        jnp.zeros((13,), jnp.float32),
    ]).astype(jnp.float32)
    h0 = _embed(x, w_exp, scal)
    agg = _round(src, dst3d, h0, scal, first=True)
    for _ in range(3):
        agg = _round(src, dst3d, agg, scal, first=False)
    hf = _finalize(agg, scal)
    return hf[:N].reshape(N, 1)
